# baseline (speedup 1.0000x reference)
import functools

import jax
import jax.numpy as jnp
from jax import lax
from jax.experimental import pallas as pl
from jax.experimental.pallas import tpu as pltpu
from jax.experimental.pallas import tpu_sc as plsc

D = 128
N0, N1, N2 = 10000, 160000, 80000
NTOT = N0 + N1 + N2

NC, NS = 2, 16
CHUNK = 4096
RPT = CHUNK // NS
K = 512
KB = K // 128
W = 2048
ZR = 64

_f32 = jnp.float32
_i32 = jnp.int32


def _pad_chunks(n):
    return ((n + CHUNK - 1) // CHUNK) * CHUNK


def _wpad(n):
    return ((n + W - 1) // W) * W


N0P, N1P, N2P = _pad_chunks(N0), _pad_chunks(N1), _pad_chunks(N2)
NNZ_L0, NNZ_I1, NNZ_L1 = _wpad(16 * N0), _wpad(2 * N1), _wpad(4 * N1)
NNZ_I2, NNZ_L2 = _wpad(3 * N2), _wpad(4 * N2)


def _mm_multi(x, ws, bn):
    n = x.shape[0]
    nw = len(ws)

    def body(x_ref, *refs):
        xv = x_ref[...]
        for wr, orf in zip(refs[:nw], refs[nw:]):
            orf[...] = jnp.dot(xv, wr[...], preferred_element_type=_f32)

    return pl.pallas_call(
        body,
        grid=(n // bn,),
        in_specs=[pl.BlockSpec((bn, D), lambda i: (i, 0))]
        + [pl.BlockSpec((D, D), lambda i: (0, 0))] * nw,
        out_specs=[pl.BlockSpec((bn, D), lambda i: (i, 0))] * nw,
        out_shape=[jax.ShapeDtypeStruct((n, D), _f32)] * nw,
    )(x, *ws)


def _ln_into(hp, x, g, b, prev, row_off, bn):
    n = x.shape[0]
    blk_off = row_off // bn
    with_prev = prev is not None

    def body(h_ref, x_ref, g_ref, b_ref, *refs):
        o_ref = refs[-1]
        s = jax.nn.sigmoid(h_ref[...]) + x_ref[...]
        mu = jnp.mean(s, axis=1, keepdims=True)
        c = s - mu
        var = jnp.mean(c * c, axis=1, keepdims=True)
        o_ref[...] = c * lax.rsqrt(var + 1e-5) * g_ref[...] + b_ref[...]

    in_specs = [
        pl.BlockSpec((bn, D), lambda i: (i, 0)),
        pl.BlockSpec((bn, D), lambda i: (i, 0)),
        pl.BlockSpec((1, D), lambda i: (0, 0)),
        pl.BlockSpec((1, D), lambda i: (0, 0)),
    ]
    args = [hp, x, g.reshape(1, D), b.reshape(1, D)]
    if with_prev:
        in_specs.append(pl.BlockSpec((bn, D), lambda i, o=blk_off: (i + o, 0)))
        args.append(prev)
    return pl.pallas_call(
        body,
        grid=(n // bn,),
        in_specs=in_specs,
        out_specs=pl.BlockSpec((bn, D), lambda i, o=blk_off: (i + o, 0)),
        out_shape=jax.ShapeDtypeStruct((NTOT, D), _f32),
        input_output_aliases={4: 0} if with_prev else {},
    )(*args)


def _sc_spmm_body(
    t_s0, t_h0, t_s1, t_l1, t_h1, t_s2, t_l2,
    l0r, l0c, l0v, i1r, i1c, i1v, l1r, l1c, l1v,
    i2r, i2c, i2v, l2r, l2c, l2v,
    h0_out, h1_out, h2_out,
    acc, rowb, colb, valb, lrow_s, col_s, val_s,
    sidx0, sidx1, sidx2, sidx3, srow0, srow1, srow2, srow3,
    gbuf, zbuf, ns_ref, sem_g, sem_w,
):
    cid = lax.axis_index("c")
    sid = lax.axis_index("s")
    sidx = (sidx0, sidx1, sidx2, sidx3)
    srow = (srow0, srow1, srow2, srow3)

    def _vcopy(src, off, dst):
        for q in range(128 // 16):
            dst[pl.ds(q * 16, 16)] = src[pl.ds(off + q * 16, 16)]

    def _zrow(r, _):
        for q in range(D // 16):
            zbuf[r, pl.ds(q * 16, 16)] = jnp.zeros((16,), _f32)
        return _

    lax.fori_loop(0, ZR, _zrow, None)

    def _zstage(i, _):
        z16i = jnp.zeros((16,), _i32)
        lrow_s[pl.ds(i * 16, 16)] = z16i
        col_s[pl.ds(i * 16, 16)] = z16i
        val_s[pl.ds(i * 16, 16)] = jnp.zeros((16,), _f32)
        return _

    lax.fori_loop(0, K // 16, _zstage, None)
    ns_ref[0] = 0

    def _zero_my_slice():
        for z in range(RPT // ZR):
            pltpu.sync_copy(zbuf, acc.at[pl.ds(sid * RPT + z * ZR, ZR)])

    _zero_my_slice()
    plsc.subcore_barrier()

    def _scale_rows(n_rows):
        iota16 = lax.iota(_i32, 16)
        n_grp = (n_rows + 15) // 16

        def grp_body(gi, _):
            vv = val_s[pl.ds(gi * 16, 16)]
            rvec = gi * 16 + iota16

            def q_body(qi, _):
                for u in range(8):
                    cvec = jnp.full((16,), qi * 8 + u, _i32)
                    col = plsc.load_gather(gbuf, [rvec, cvec])
                    plsc.store_scatter(gbuf, [rvec, cvec], col * vv)
                return _

            lax.fori_loop(0, D // 8, q_body, None)
            return _

        lax.fori_loop(0, n_grp, grp_body, None)

    def _reset_stage():
        def zb(i, _):
            val_s[pl.ds(i * 16, 16)] = jnp.zeros((16,), _f32)
            return _

        lax.fori_loop(0, K // 16, zb, None)
        ns_ref[0] = 0

    def _flush_full(tbl, acc_ref):
        descs = []
        for j in range(KB):
            _vcopy(col_s, j * 128, sidx[j])
            descs.append(
                pltpu.async_copy(
                    tbl.at[sidx[j]], gbuf.at[pl.ds(j * 128, 128)], sem_g
                )
            )
        for d in descs:
            d.wait()
        _scale_rows(K)
        for j in range(KB):
            _vcopy(lrow_s, j * 128, srow[j])
            pltpu.sync_copy(
                gbuf.at[pl.ds(j * 128, 128)], acc_ref.at[srow[j]], add=True
            )
        _reset_stage()

    def _flush_tail(tbl, acc_ref):
        n = ns_ref[0]

        @pl.when(n > 0)
        def _():
            for j in range(KB):
                @pl.when(n > j * 128)
                def _():
                    _vcopy(col_s, j * 128, sidx[j])
                    pltpu.sync_copy(
                        tbl.at[sidx[j]], gbuf.at[pl.ds(j * 128, 128)]
                    )
            _scale_rows(((n + 127) // 128) * 128)
            for j in range(KB):
                @pl.when(n > j * 128)
                def _():
                    _vcopy(lrow_s, j * 128, srow[j])
                    pltpu.sync_copy(
                        gbuf.at[pl.ds(j * 128, 128)],
                        acc_ref.at[srow[j]],
                        add=True,
                    )
            _reset_stage()

    def _scan_vregs(row0, n_vregs, tbl, acc_ref):
        def vbody(v, _):
            r = rowb[pl.ds(v * 16, 16)]
            cvec = colb[pl.ds(v * 16, 16)]
            vvec = valb[pl.ds(v * 16, 16)]
            m = (r >= row0) & (r < row0 + CHUNK)
            ns = ns_ref[0]
            s = plsc.cumsum(jnp.where(m, 1, 0).astype(_i32))
            pos = s + (ns - 1)
            plsc.store_scatter(lrow_s, [pos], r - row0, mask=m)
            plsc.store_scatter(col_s, [pos], cvec, mask=m)
            plsc.store_scatter(val_s, [pos], vvec, mask=m)
            ns_ref[0] = ns + s[15]

            @pl.when(ns_ref[0] >= K - 16)
            def _():
                _flush_full(tbl, acc_ref)

            return _

        lax.fori_loop(0, n_vregs, vbody, None)

    def _process_triplet(row0, acc_ref, rr, cc, vv, nnz, tbl):
        full_w = nnz // W
        n_win_me = (full_w - sid + 15) // 16

        def win_body(k, _):
            off = (k * 16 + sid) * W
            d0 = pltpu.async_copy(rr.at[pl.ds(off, W)], rowb, sem_w)
            d1 = pltpu.async_copy(cc.at[pl.ds(off, W)], colb, sem_w)
            d2 = pltpu.async_copy(vv.at[pl.ds(off, W)], valb, sem_w)
            d0.wait()
            d1.wait()
            d2.wait()
            _scan_vregs(row0, W // 16, tbl, acc_ref)
            return _

        lax.fori_loop(0, n_win_me, win_body, None)
        _flush_tail(tbl, acc_ref)

    def _process_rank(out_hbm, n_pad, triplets):
        c_total = n_pad // CHUNK
        c0 = (c_total + 1) // 2
        n_me = jnp.where(cid == 0, c0, c_total - c0)
        base_c = jnp.where(cid == 0, 0, c0)
        n_max = c0

        def chunk_body(i, _):
            @pl.when(i < n_me)
            def _():
                c = base_c + i
                row0 = c * CHUNK
                for (rr, cc, vv, nnz, tbl) in triplets:
                    _process_triplet(row0, acc, rr, cc, vv, nnz, tbl)
                plsc.subcore_barrier()
                pltpu.sync_copy(
                    acc.at[pl.ds(sid * RPT, RPT)],
                    out_hbm.at[pl.ds(row0 + sid * RPT, RPT)],
                )
                _zero_my_slice()
                plsc.subcore_barrier()

            return _

        lax.fori_loop(0, n_max, chunk_body, None)

    _process_rank(
        h0_out, N0P,
        [(l0r, l0c, l0v, NNZ_L0, t_s0), (i1r, i1c, i1v, NNZ_I1, t_h0)],
    )
    _process_rank(
        h1_out, N1P,
        [
            (l1r, l1c, l1v, NNZ_L1, t_s1),
            (i1c, i1r, i1v, NNZ_I1, t_l1),
            (i2r, i2c, i2v, NNZ_I2, t_h1),
        ],
    )
    _process_rank(
        h2_out, N2P,
        [(l2r, l2c, l2v, NNZ_L2, t_s2), (i2c, i2r, i2v, NNZ_I2, t_l2)],
    )


_sc_spmm = pl.kernel(
    _sc_spmm_body,
    out_type=[
        jax.ShapeDtypeStruct((N0P, D), _f32),
        jax.ShapeDtypeStruct((N1P, D), _f32),
        jax.ShapeDtypeStruct((N2P, D), _f32),
    ],
    mesh=plsc.VectorSubcoreMesh(
        core_axis_name="c", subcore_axis_name="s", num_cores=NC, num_subcores=NS
    ),
    compiler_params=pltpu.CompilerParams(needs_layout_passes=False),
    scratch_types=[
        pltpu.VMEM_SHARED((CHUNK, D), _f32),
        pltpu.VMEM((W,), _i32),
        pltpu.VMEM((W,), _i32),
        pltpu.VMEM((W,), _f32),
        pltpu.VMEM((K,), _i32),
        pltpu.VMEM((K,), _i32),
        pltpu.VMEM((K,), _f32),
        pltpu.VMEM((128,), _i32),
        pltpu.VMEM((128,), _i32),
        pltpu.VMEM((128,), _i32),
        pltpu.VMEM((128,), _i32),
        pltpu.VMEM((128,), _i32),
        pltpu.VMEM((128,), _i32),
        pltpu.VMEM((128,), _i32),
        pltpu.VMEM((128,), _i32),
        pltpu.VMEM((K, D), _f32),
        pltpu.VMEM((ZR, D), _f32),
        pltpu.SMEM((1,), _i32),
        pltpu.SemaphoreType.DMA,
        pltpu.SemaphoreType.DMA,
    ],
)


def kernel(x_0, x_1, x_2, inc1_rows, inc1_cols, inc1_vals, inc2_rows, inc2_cols, inc2_vals, L0_rows, L0_cols, L0_vals, L1_rows, L1_cols, L1_vals, L2_rows, L2_cols, L2_vals, W_same_0, W_same_1, W_same_2, W_low_1, W_low_2, W_high_0, W_high_1, g0, b0, g1, b1, g2, b2, y, batch_0):
    t_s0, t_l1 = _mm_multi(x_0, [W_same_0, W_low_1], 2000)
    t_s1, t_h0, t_l2 = _mm_multi(x_1, [W_same_1, W_high_0, W_low_2], 2000)
    t_s2, t_h1 = _mm_multi(x_2, [W_same_2, W_high_1], 2000)

    def _pad_coo(r, c, v):
        npad = _wpad(r.shape[0]) - r.shape[0]
        if npad == 0:
            return r, c, v
        return (
            jnp.concatenate([r, jnp.full((npad,), -1, r.dtype)]),
            jnp.concatenate([c, jnp.full((npad,), -1, c.dtype)]),
            jnp.concatenate([v, jnp.zeros((npad,), v.dtype)]),
        )

    l0 = _pad_coo(L0_rows, L0_cols, L0_vals)
    i1 = _pad_coo(inc1_rows, inc1_cols, inc1_vals)
    l1 = _pad_coo(L1_rows, L1_cols, L1_vals)
    i2 = _pad_coo(inc2_rows, inc2_cols, inc2_vals)
    l2 = _pad_coo(L2_rows, L2_cols, L2_vals)

    h0p, h1p, h2p = _sc_spmm(
        t_s0, t_h0, t_s1, t_l1, t_h1, t_s2, t_l2,
        *l0, *i1, *l1, *i2, *l2,
    )

    out = _ln_into(h0p, x_0, g0, b0, None, 0, 2000)
    out = _ln_into(h1p, x_1, g1, b1, out, N0, 2000)
    out = _ln_into(h2p, x_2, g2, b2, out, N0 + N1, 2000)
    return out


# speedup vs baseline: 2.0792x; 2.0792x over previous
import functools

import jax
import jax.numpy as jnp
from jax import lax
from jax.experimental import pallas as pl
from jax.experimental.pallas import tpu as pltpu
from jax.experimental.pallas import tpu_sc as plsc

D = 128
N0, N1, N2 = 10000, 160000, 80000
NTOT = N0 + N1 + N2

NC, NS = 2, 16
CHUNK = 4096
RPT = CHUNK // NS
K = 512
KB = K // 128
W = 2048
ZR = 64

_f32 = jnp.float32
_i32 = jnp.int32


def _pad_chunks(n):
    return ((n + CHUNK - 1) // CHUNK) * CHUNK


def _wpad(n):
    return ((n + W - 1) // W) * W


N0P, N1P, N2P = _pad_chunks(N0), _pad_chunks(N1), _pad_chunks(N2)
NNZ_L0, NNZ_I1, NNZ_L1 = _wpad(16 * N0), _wpad(2 * N1), _wpad(4 * N1)
NNZ_I2, NNZ_L2 = _wpad(3 * N2), _wpad(4 * N2)


def _mm_multi(x, ws, bn):
    n = x.shape[0]
    nw = len(ws)

    def body(x_ref, *refs):
        xv = x_ref[...]
        for wr, orf in zip(refs[:nw], refs[nw:]):
            orf[...] = jnp.dot(xv, wr[...], preferred_element_type=_f32)

    return pl.pallas_call(
        body,
        grid=(n // bn,),
        in_specs=[pl.BlockSpec((bn, D), lambda i: (i, 0))]
        + [pl.BlockSpec((D, D), lambda i: (0, 0))] * nw,
        out_specs=[pl.BlockSpec((bn, D), lambda i: (i, 0))] * nw,
        out_shape=[jax.ShapeDtypeStruct((n, D), _f32)] * nw,
    )(x, *ws)


def _ln_into(hp, x, g, b, prev, row_off, bn):
    n = x.shape[0]
    blk_off = row_off // bn
    with_prev = prev is not None

    def body(h_ref, x_ref, g_ref, b_ref, *refs):
        o_ref = refs[-1]
        s = jax.nn.sigmoid(h_ref[...]) + x_ref[...]
        mu = jnp.mean(s, axis=1, keepdims=True)
        c = s - mu
        var = jnp.mean(c * c, axis=1, keepdims=True)
        o_ref[...] = c * lax.rsqrt(var + 1e-5) * g_ref[...] + b_ref[...]

    in_specs = [
        pl.BlockSpec((bn, D), lambda i: (i, 0)),
        pl.BlockSpec((bn, D), lambda i: (i, 0)),
        pl.BlockSpec((1, D), lambda i: (0, 0)),
        pl.BlockSpec((1, D), lambda i: (0, 0)),
    ]
    args = [hp, x, g.reshape(1, D), b.reshape(1, D)]
    if with_prev:
        in_specs.append(pl.BlockSpec((bn, D), lambda i, o=blk_off: (i + o, 0)))
        args.append(prev)
    return pl.pallas_call(
        body,
        grid=(n // bn,),
        in_specs=in_specs,
        out_specs=pl.BlockSpec((bn, D), lambda i, o=blk_off: (i + o, 0)),
        out_shape=jax.ShapeDtypeStruct((NTOT, D), _f32),
        input_output_aliases={4: 0} if with_prev else {},
    )(*args)


def _sc_spmm_body(
    t_s0, t_h0, t_s1, t_l1, t_h1, t_s2, t_l2,
    l0r, l0c, l0v, i1r, i1c, i1v, l1r, l1c, l1v,
    i2r, i2c, i2v, l2r, l2c, l2v,
    h0_out, h1_out, h2_out,
    acc, rowb, colb, valb, lrow_s, col_s, val_s,
    sidx0, sidx1, sidx2, sidx3, srow0, srow1, srow2, srow3,
    gbuf, zbuf, ns_ref, sem_g, sem_w,
):
    cid = lax.axis_index("c")
    sid = lax.axis_index("s")
    sidx = (sidx0, sidx1, sidx2, sidx3)
    srow = (srow0, srow1, srow2, srow3)

    def _vcopy(src, off, dst):
        for q in range(128 // 16):
            dst[pl.ds(q * 16, 16)] = src[pl.ds(off + q * 16, 16)]

    def _zrow(r, _):
        for q in range(D // 16):
            zbuf[r, pl.ds(q * 16, 16)] = jnp.zeros((16,), _f32)
        return _

    lax.fori_loop(0, ZR, _zrow, None)

    def _zstage(i, _):
        z16i = jnp.zeros((16,), _i32)
        lrow_s[pl.ds(i * 16, 16)] = z16i
        col_s[pl.ds(i * 16, 16)] = z16i
        val_s[pl.ds(i * 16, 16)] = jnp.zeros((16,), _f32)
        return _

    lax.fori_loop(0, K // 16, _zstage, None)
    ns_ref[0] = 0

    def _zero_my_slice():
        for z in range(RPT // ZR):
            pltpu.sync_copy(zbuf, acc.at[pl.ds(sid * RPT + z * ZR, ZR)])

    _zero_my_slice()
    plsc.subcore_barrier()

    def _scale_rows(n_rows):
        iota16 = lax.iota(_i32, 16)
        n_grp = (n_rows + 15) // 16

        def grp_body(gi, _):
            vv = val_s[pl.ds(gi * 16, 16)]
            rvec = gi * 16 + iota16

            def q_body(qi, _):
                for u in range(8):
                    cvec = (iota16 + (qi * 8 + u)) & (D - 1)
                    col = plsc.load_gather(gbuf, [rvec, cvec])
                    plsc.store_scatter(gbuf, [rvec, cvec], col * vv)
                return _

            lax.fori_loop(0, D // 8, q_body, None)
            return _

        lax.fori_loop(0, n_grp, grp_body, None)

    def _reset_stage():
        def zb(i, _):
            val_s[pl.ds(i * 16, 16)] = jnp.zeros((16,), _f32)
            return _

        lax.fori_loop(0, K // 16, zb, None)
        ns_ref[0] = 0

    def _flush_full(tbl, acc_ref):
        descs = []
        for j in range(KB):
            _vcopy(col_s, j * 128, sidx[j])
            descs.append(
                pltpu.async_copy(
                    tbl.at[sidx[j]], gbuf.at[pl.ds(j * 128, 128)], sem_g
                )
            )
        for d in descs:
            d.wait()
        _scale_rows(K)
        for j in range(KB):
            _vcopy(lrow_s, j * 128, srow[j])
            pltpu.sync_copy(
                gbuf.at[pl.ds(j * 128, 128)], acc_ref.at[srow[j]], add=True
            )
        _reset_stage()

    def _flush_tail(tbl, acc_ref):
        n = ns_ref[0]

        @pl.when(n > 0)
        def _():
            for j in range(KB):
                @pl.when(n > j * 128)
                def _():
                    _vcopy(col_s, j * 128, sidx[j])
                    pltpu.sync_copy(
                        tbl.at[sidx[j]], gbuf.at[pl.ds(j * 128, 128)]
                    )
            _scale_rows(((n + 127) // 128) * 128)
            for j in range(KB):
                @pl.when(n > j * 128)
                def _():
                    _vcopy(lrow_s, j * 128, srow[j])
                    pltpu.sync_copy(
                        gbuf.at[pl.ds(j * 128, 128)],
                        acc_ref.at[srow[j]],
                        add=True,
                    )
            _reset_stage()

    def _scan_vregs(row0, n_vregs, tbl, acc_ref):
        def vbody(v, _):
            r = rowb[pl.ds(v * 16, 16)]
            cvec = colb[pl.ds(v * 16, 16)]
            vvec = valb[pl.ds(v * 16, 16)]
            m = (r >= row0) & (r < row0 + CHUNK)
            ns = ns_ref[0]
            s = plsc.cumsum(jnp.where(m, 1, 0).astype(_i32))
            pos = s + (ns - 1)
            plsc.store_scatter(lrow_s, [pos], r - row0, mask=m)
            plsc.store_scatter(col_s, [pos], cvec, mask=m)
            plsc.store_scatter(val_s, [pos], vvec, mask=m)
            ns_ref[0] = ns + s[15]

            @pl.when(ns_ref[0] >= K - 16)
            def _():
                _flush_full(tbl, acc_ref)

            return _

        lax.fori_loop(0, n_vregs, vbody, None)

    def _process_triplet(row0, acc_ref, rr, cc, vv, nnz, tbl):
        full_w = nnz // W
        n_win_me = (full_w - sid + 15) // 16

        def win_body(k, _):
            off = (k * 16 + sid) * W
            d0 = pltpu.async_copy(rr.at[pl.ds(off, W)], rowb, sem_w)
            d1 = pltpu.async_copy(cc.at[pl.ds(off, W)], colb, sem_w)
            d2 = pltpu.async_copy(vv.at[pl.ds(off, W)], valb, sem_w)
            d0.wait()
            d1.wait()
            d2.wait()
            _scan_vregs(row0, W // 16, tbl, acc_ref)
            return _

        lax.fori_loop(0, n_win_me, win_body, None)
        _flush_tail(tbl, acc_ref)

    def _process_rank(out_hbm, n_pad, triplets):
        c_total = n_pad // CHUNK
        c0 = (c_total + 1) // 2
        n_me = jnp.where(cid == 0, c0, c_total - c0)
        base_c = jnp.where(cid == 0, 0, c0)
        n_max = c0

        def chunk_body(i, _):
            @pl.when(i < n_me)
            def _():
                c = base_c + i
                row0 = c * CHUNK
                for (rr, cc, vv, nnz, tbl) in triplets:
                    _process_triplet(row0, acc, rr, cc, vv, nnz, tbl)
                plsc.subcore_barrier()
                pltpu.sync_copy(
                    acc.at[pl.ds(sid * RPT, RPT)],
                    out_hbm.at[pl.ds(row0 + sid * RPT, RPT)],
                )
                _zero_my_slice()
                plsc.subcore_barrier()

            return _

        lax.fori_loop(0, n_max, chunk_body, None)

    _process_rank(
        h0_out, N0P,
        [(l0r, l0c, l0v, NNZ_L0, t_s0), (i1r, i1c, i1v, NNZ_I1, t_h0)],
    )
    _process_rank(
        h1_out, N1P,
        [
            (l1r, l1c, l1v, NNZ_L1, t_s1),
            (i1c, i1r, i1v, NNZ_I1, t_l1),
            (i2r, i2c, i2v, NNZ_I2, t_h1),
        ],
    )
    _process_rank(
        h2_out, N2P,
        [(l2r, l2c, l2v, NNZ_L2, t_s2), (i2c, i2r, i2v, NNZ_I2, t_l2)],
    )


_sc_spmm = pl.kernel(
    _sc_spmm_body,
    out_type=[
        jax.ShapeDtypeStruct((N0P, D), _f32),
        jax.ShapeDtypeStruct((N1P, D), _f32),
        jax.ShapeDtypeStruct((N2P, D), _f32),
    ],
    mesh=plsc.VectorSubcoreMesh(
        core_axis_name="c", subcore_axis_name="s", num_cores=NC, num_subcores=NS
    ),
    compiler_params=pltpu.CompilerParams(needs_layout_passes=False),
    scratch_types=[
        pltpu.VMEM_SHARED((CHUNK, D), _f32),
        pltpu.VMEM((W,), _i32),
        pltpu.VMEM((W,), _i32),
        pltpu.VMEM((W,), _f32),
        pltpu.VMEM((K,), _i32),
        pltpu.VMEM((K,), _i32),
        pltpu.VMEM((K,), _f32),
        pltpu.VMEM((128,), _i32),
        pltpu.VMEM((128,), _i32),
        pltpu.VMEM((128,), _i32),
        pltpu.VMEM((128,), _i32),
        pltpu.VMEM((128,), _i32),
        pltpu.VMEM((128,), _i32),
        pltpu.VMEM((128,), _i32),
        pltpu.VMEM((128,), _i32),
        pltpu.VMEM((K, D), _f32),
        pltpu.VMEM((ZR, D), _f32),
        pltpu.SMEM((1,), _i32),
        pltpu.SemaphoreType.DMA,
        pltpu.SemaphoreType.DMA,
    ],
)


def kernel(x_0, x_1, x_2, inc1_rows, inc1_cols, inc1_vals, inc2_rows, inc2_cols, inc2_vals, L0_rows, L0_cols, L0_vals, L1_rows, L1_cols, L1_vals, L2_rows, L2_cols, L2_vals, W_same_0, W_same_1, W_same_2, W_low_1, W_low_2, W_high_0, W_high_1, g0, b0, g1, b1, g2, b2, y, batch_0):
    t_s0, t_l1 = _mm_multi(x_0, [W_same_0, W_low_1], 2000)
    t_s1, t_h0, t_l2 = _mm_multi(x_1, [W_same_1, W_high_0, W_low_2], 2000)
    t_s2, t_h1 = _mm_multi(x_2, [W_same_2, W_high_1], 2000)

    def _pad_coo(r, c, v):
        npad = _wpad(r.shape[0]) - r.shape[0]
        if npad == 0:
            return r, c, v
        return (
            jnp.concatenate([r, jnp.full((npad,), -1, r.dtype)]),
            jnp.concatenate([c, jnp.full((npad,), -1, c.dtype)]),
            jnp.concatenate([v, jnp.zeros((npad,), v.dtype)]),
        )

    l0 = _pad_coo(L0_rows, L0_cols, L0_vals)
    i1 = _pad_coo(inc1_rows, inc1_cols, inc1_vals)
    l1 = _pad_coo(L1_rows, L1_cols, L1_vals)
    i2 = _pad_coo(inc2_rows, inc2_cols, inc2_vals)
    l2 = _pad_coo(L2_rows, L2_cols, L2_vals)

    h0p, h1p, h2p = _sc_spmm(
        t_s0, t_h0, t_s1, t_l1, t_h1, t_s2, t_l2,
        *l0, *i1, *l1, *i2, *l2,
    )

    out = _ln_into(h0p, x_0, g0, b0, None, 0, 2000)
    out = _ln_into(h1p, x_1, g1, b1, out, N0, 2000)
    out = _ln_into(h2p, x_2, g2, b2, out, N0 + N1, 2000)
    return out


# speedup vs baseline: 2.4228x; 1.1653x over previous
import functools

import jax
import jax.numpy as jnp
from jax import lax
from jax.experimental import pallas as pl
from jax.experimental.pallas import tpu as pltpu
from jax.experimental.pallas import tpu_sc as plsc

D = 128
N0, N1, N2 = 10000, 160000, 80000
NTOT = N0 + N1 + N2

NC, NS = 2, 16
CHUNK = 6144
RPT = CHUNK // NS
K = 512
KB = K // 128
W = 2048
ZR = 32

_f32 = jnp.float32
_i32 = jnp.int32


def _pad_chunks(n):
    return ((n + CHUNK - 1) // CHUNK) * CHUNK


def _wpad(n):
    return ((n + W - 1) // W) * W


N0P, N1P, N2P = _pad_chunks(N0), _pad_chunks(N1), _pad_chunks(N2)
NNZ_L0, NNZ_I1, NNZ_L1 = _wpad(16 * N0), _wpad(2 * N1), _wpad(4 * N1)
NNZ_I2, NNZ_L2 = _wpad(3 * N2), _wpad(4 * N2)


def _mm_multi(x, ws, bn):
    n = x.shape[0]
    nw = len(ws)

    def body(x_ref, *refs):
        xv = x_ref[...]
        for wr, orf in zip(refs[:nw], refs[nw:]):
            orf[...] = jnp.dot(xv, wr[...], preferred_element_type=_f32)

    return pl.pallas_call(
        body,
        grid=(n // bn,),
        in_specs=[pl.BlockSpec((bn, D), lambda i: (i, 0))]
        + [pl.BlockSpec((D, D), lambda i: (0, 0))] * nw,
        out_specs=[pl.BlockSpec((bn, D), lambda i: (i, 0))] * nw,
        out_shape=[jax.ShapeDtypeStruct((n, D), _f32)] * nw,
    )(x, *ws)


def _ln_into(hp, x, g, b, prev, row_off, bn):
    n = x.shape[0]
    blk_off = row_off // bn
    with_prev = prev is not None

    def body(h_ref, x_ref, g_ref, b_ref, *refs):
        o_ref = refs[-1]
        s = jax.nn.sigmoid(h_ref[...]) + x_ref[...]
        mu = jnp.mean(s, axis=1, keepdims=True)
        c = s - mu
        var = jnp.mean(c * c, axis=1, keepdims=True)
        o_ref[...] = c * lax.rsqrt(var + 1e-5) * g_ref[...] + b_ref[...]

    in_specs = [
        pl.BlockSpec((bn, D), lambda i: (i, 0)),
        pl.BlockSpec((bn, D), lambda i: (i, 0)),
        pl.BlockSpec((1, D), lambda i: (0, 0)),
        pl.BlockSpec((1, D), lambda i: (0, 0)),
    ]
    args = [hp, x, g.reshape(1, D), b.reshape(1, D)]
    if with_prev:
        in_specs.append(pl.BlockSpec((bn, D), lambda i, o=blk_off: (i + o, 0)))
        args.append(prev)
    return pl.pallas_call(
        body,
        grid=(n // bn,),
        in_specs=in_specs,
        out_specs=pl.BlockSpec((bn, D), lambda i, o=blk_off: (i + o, 0)),
        out_shape=jax.ShapeDtypeStruct((NTOT, D), _f32),
        input_output_aliases={4: 0} if with_prev else {},
    )(*args)


def _sc_spmm_body(
    t_s0, t_h0, t_s1, t_l1, t_h1, t_s2, t_l2,
    l0r, l0c, l0v, i1r, i1c, i1v, l1r, l1c, l1v,
    i2r, i2c, i2v, l2r, l2c, l2v,
    h0_out, h1_out, h2_out,
    acc, rowb, colb, valb, lrow_s, col_s, val_s,
    sidx0, sidx1, sidx2, sidx3, srow0, srow1, srow2, srow3,
    gbuf, zbuf, ns_ref, sem_g, sem_w,
):
    cid = lax.axis_index("c")
    sid = lax.axis_index("s")
    sidx = (sidx0, sidx1, sidx2, sidx3)
    srow = (srow0, srow1, srow2, srow3)

    def _vcopy(src, off, dst):
        for q in range(128 // 16):
            dst[pl.ds(q * 16, 16)] = src[pl.ds(off + q * 16, 16)]

    def _zrow(r, _):
        for q in range(D // 16):
            zbuf[r, pl.ds(q * 16, 16)] = jnp.zeros((16,), _f32)
        return _

    lax.fori_loop(0, ZR, _zrow, None)

    def _zstage(i, _):
        z16i = jnp.zeros((16,), _i32)
        lrow_s[pl.ds(i * 16, 16)] = z16i
        col_s[pl.ds(i * 16, 16)] = z16i
        val_s[pl.ds(i * 16, 16)] = jnp.zeros((16,), _f32)
        return _

    lax.fori_loop(0, K // 16, _zstage, None)
    ns_ref[0] = 0

    def _zero_my_slice():
        for z in range(RPT // ZR):
            pltpu.sync_copy(zbuf, acc.at[pl.ds(sid * RPT + z * ZR, ZR)])

    _zero_my_slice()
    plsc.subcore_barrier()

    def _scale_rows(n_rows):
        iota16 = lax.iota(_i32, 16)
        n_grp = (n_rows + 15) // 16

        def grp_body(gi, _):
            vv = val_s[pl.ds(gi * 16, 16)]
            rvec = gi * 16 + iota16

            def q_body(qi, _):
                for u in range(8):
                    cvec = (iota16 + (qi * 8 + u)) & (D - 1)
                    col = plsc.load_gather(gbuf, [rvec, cvec])
                    plsc.store_scatter(gbuf, [rvec, cvec], col * vv)
                return _

            lax.fori_loop(0, D // 8, q_body, None)
            return _

        lax.fori_loop(0, n_grp, grp_body, None)

    def _reset_stage():
        def zb(i, _):
            val_s[pl.ds(i * 16, 16)] = jnp.zeros((16,), _f32)
            return _

        lax.fori_loop(0, K // 16, zb, None)
        ns_ref[0] = 0

    def _flush_full(tbl, acc_ref):
        descs = []
        for j in range(KB):
            _vcopy(col_s, j * 128, sidx[j])
            descs.append(
                pltpu.async_copy(
                    tbl.at[sidx[j]], gbuf.at[pl.ds(j * 128, 128)], sem_g
                )
            )
        for d in descs:
            d.wait()
        _scale_rows(K)
        for j in range(KB):
            _vcopy(lrow_s, j * 128, srow[j])
            pltpu.sync_copy(
                gbuf.at[pl.ds(j * 128, 128)], acc_ref.at[srow[j]], add=True
            )
        _reset_stage()

    def _flush_tail(tbl, acc_ref):
        n = ns_ref[0]

        @pl.when(n > 0)
        def _():
            for j in range(KB):
                @pl.when(n > j * 128)
                def _():
                    _vcopy(col_s, j * 128, sidx[j])
                    pltpu.sync_copy(
                        tbl.at[sidx[j]], gbuf.at[pl.ds(j * 128, 128)]
                    )
            _scale_rows(((n + 127) // 128) * 128)
            for j in range(KB):
                @pl.when(n > j * 128)
                def _():
                    _vcopy(lrow_s, j * 128, srow[j])
                    pltpu.sync_copy(
                        gbuf.at[pl.ds(j * 128, 128)],
                        acc_ref.at[srow[j]],
                        add=True,
                    )
            _reset_stage()

    def _scan_vregs(row0, n_vregs, tbl, acc_ref):
        def vbody(v, _):
            r = rowb[pl.ds(v * 16, 16)]
            cvec = colb[pl.ds(v * 16, 16)]
            vvec = valb[pl.ds(v * 16, 16)]
            m = (r >= row0) & (r < row0 + CHUNK)
            cnt = plsc.all_reduce_population_count(m)

            @pl.when(cnt[0] > 0)
            def _():
                ns = ns_ref[0]
                s = plsc.cumsum(jnp.where(m, 1, 0).astype(_i32))
                pos = s + (ns - 1)
                plsc.store_scatter(lrow_s, [pos], r - row0, mask=m)
                plsc.store_scatter(col_s, [pos], cvec, mask=m)
                plsc.store_scatter(val_s, [pos], vvec, mask=m)
                ns_ref[0] = ns + cnt[0]

                @pl.when(ns_ref[0] >= K - 16)
                def _():
                    _flush_full(tbl, acc_ref)

            return _

        lax.fori_loop(0, n_vregs, vbody, None)

    def _process_triplet(row0, acc_ref, rr, cc, vv, nnz, tbl):
        full_w = nnz // W
        n_win_me = (full_w - sid + 15) // 16

        def win_body(k, _):
            off = (k * 16 + sid) * W
            d0 = pltpu.async_copy(rr.at[pl.ds(off, W)], rowb, sem_w)
            d1 = pltpu.async_copy(cc.at[pl.ds(off, W)], colb, sem_w)
            d2 = pltpu.async_copy(vv.at[pl.ds(off, W)], valb, sem_w)
            d0.wait()
            d1.wait()
            d2.wait()
            _scan_vregs(row0, W // 16, tbl, acc_ref)
            return _

        lax.fori_loop(0, n_win_me, win_body, None)
        _flush_tail(tbl, acc_ref)

    def _process_rank(out_hbm, n_pad, triplets):
        c_total = n_pad // CHUNK
        c0 = (c_total + 1) // 2
        n_me = jnp.where(cid == 0, c0, c_total - c0)
        base_c = jnp.where(cid == 0, 0, c0)
        n_max = c0

        def chunk_body(i, _):
            @pl.when(i < n_me)
            def _():
                c = base_c + i
                row0 = c * CHUNK
                for (rr, cc, vv, nnz, tbl) in triplets:
                    _process_triplet(row0, acc, rr, cc, vv, nnz, tbl)
                plsc.subcore_barrier()
                pltpu.sync_copy(
                    acc.at[pl.ds(sid * RPT, RPT)],
                    out_hbm.at[pl.ds(row0 + sid * RPT, RPT)],
                )
                _zero_my_slice()
                plsc.subcore_barrier()

            return _

        lax.fori_loop(0, n_max, chunk_body, None)

    _process_rank(
        h0_out, N0P,
        [(l0r, l0c, l0v, NNZ_L0, t_s0), (i1r, i1c, i1v, NNZ_I1, t_h0)],
    )
    _process_rank(
        h1_out, N1P,
        [
            (l1r, l1c, l1v, NNZ_L1, t_s1),
            (i1c, i1r, i1v, NNZ_I1, t_l1),
            (i2r, i2c, i2v, NNZ_I2, t_h1),
        ],
    )
    _process_rank(
        h2_out, N2P,
        [(l2r, l2c, l2v, NNZ_L2, t_s2), (i2c, i2r, i2v, NNZ_I2, t_l2)],
    )


_sc_spmm = pl.kernel(
    _sc_spmm_body,
    out_type=[
        jax.ShapeDtypeStruct((N0P, D), _f32),
        jax.ShapeDtypeStruct((N1P, D), _f32),
        jax.ShapeDtypeStruct((N2P, D), _f32),
    ],
    mesh=plsc.VectorSubcoreMesh(
        core_axis_name="c", subcore_axis_name="s", num_cores=NC, num_subcores=NS
    ),
    compiler_params=pltpu.CompilerParams(needs_layout_passes=False),
    scratch_types=[
        pltpu.VMEM_SHARED((CHUNK, D), _f32),
        pltpu.VMEM((W,), _i32),
        pltpu.VMEM((W,), _i32),
        pltpu.VMEM((W,), _f32),
        pltpu.VMEM((K,), _i32),
        pltpu.VMEM((K,), _i32),
        pltpu.VMEM((K,), _f32),
        pltpu.VMEM((128,), _i32),
        pltpu.VMEM((128,), _i32),
        pltpu.VMEM((128,), _i32),
        pltpu.VMEM((128,), _i32),
        pltpu.VMEM((128,), _i32),
        pltpu.VMEM((128,), _i32),
        pltpu.VMEM((128,), _i32),
        pltpu.VMEM((128,), _i32),
        pltpu.VMEM((K, D), _f32),
        pltpu.VMEM((ZR, D), _f32),
        pltpu.SMEM((1,), _i32),
        pltpu.SemaphoreType.DMA,
        pltpu.SemaphoreType.DMA,
    ],
)


def kernel(x_0, x_1, x_2, inc1_rows, inc1_cols, inc1_vals, inc2_rows, inc2_cols, inc2_vals, L0_rows, L0_cols, L0_vals, L1_rows, L1_cols, L1_vals, L2_rows, L2_cols, L2_vals, W_same_0, W_same_1, W_same_2, W_low_1, W_low_2, W_high_0, W_high_1, g0, b0, g1, b1, g2, b2, y, batch_0):
    t_s0, t_l1 = _mm_multi(x_0, [W_same_0, W_low_1], 2000)
    t_s1, t_h0, t_l2 = _mm_multi(x_1, [W_same_1, W_high_0, W_low_2], 2000)
    t_s2, t_h1 = _mm_multi(x_2, [W_same_2, W_high_1], 2000)

    def _pad_coo(r, c, v):
        npad = _wpad(r.shape[0]) - r.shape[0]
        if npad == 0:
            return r, c, v
        return (
            jnp.concatenate([r, jnp.full((npad,), -1, r.dtype)]),
            jnp.concatenate([c, jnp.full((npad,), -1, c.dtype)]),
            jnp.concatenate([v, jnp.zeros((npad,), v.dtype)]),
        )

    l0 = _pad_coo(L0_rows, L0_cols, L0_vals)
    i1 = _pad_coo(inc1_rows, inc1_cols, inc1_vals)
    l1 = _pad_coo(L1_rows, L1_cols, L1_vals)
    i2 = _pad_coo(inc2_rows, inc2_cols, inc2_vals)
    l2 = _pad_coo(L2_rows, L2_cols, L2_vals)

    h0p, h1p, h2p = _sc_spmm(
        t_s0, t_h0, t_s1, t_l1, t_h1, t_s2, t_l2,
        *l0, *i1, *l1, *i2, *l2,
    )

    out = _ln_into(h0p, x_0, g0, b0, None, 0, 2000)
    out = _ln_into(h1p, x_1, g1, b1, out, N0, 2000)
    out = _ln_into(h2p, x_2, g2, b2, out, N0 + N1, 2000)
    return out


# speedup vs baseline: 2.5092x; 1.0356x over previous
import functools

import jax
import jax.numpy as jnp
from jax import lax
from jax.experimental import pallas as pl
from jax.experimental.pallas import tpu as pltpu
from jax.experimental.pallas import tpu_sc as plsc

D = 128
N0, N1, N2 = 10000, 160000, 80000
NTOT = N0 + N1 + N2

NC, NS = 2, 16
CHUNK = 6144
RPT = CHUNK // NS
K = 512
KB = K // 128
W = 2048
ZR = 32

_f32 = jnp.float32
_i32 = jnp.int32


def _pad_chunks(n):
    return ((n + CHUNK - 1) // CHUNK) * CHUNK


def _wpad(n):
    return ((n + W - 1) // W) * W


N0P, N1P, N2P = _pad_chunks(N0), _pad_chunks(N1), _pad_chunks(N2)
NNZ_L0, NNZ_I1, NNZ_L1 = _wpad(16 * N0), _wpad(2 * N1), _wpad(4 * N1)
NNZ_I2, NNZ_L2 = _wpad(3 * N2), _wpad(4 * N2)


def _mm_multi(x, ws, bn):
    n = x.shape[0]
    nw = len(ws)

    def body(x_ref, *refs):
        xv = x_ref[...]
        for wr, orf in zip(refs[:nw], refs[nw:]):
            orf[...] = jnp.dot(xv, wr[...], preferred_element_type=_f32)

    return pl.pallas_call(
        body,
        grid=(n // bn,),
        in_specs=[pl.BlockSpec((bn, D), lambda i: (i, 0))]
        + [pl.BlockSpec((D, D), lambda i: (0, 0))] * nw,
        out_specs=[pl.BlockSpec((bn, D), lambda i: (i, 0))] * nw,
        out_shape=[jax.ShapeDtypeStruct((n, D), _f32)] * nw,
    )(x, *ws)


def _ln_into(hp, x, g, b, prev, row_off, bn):
    n = x.shape[0]
    blk_off = row_off // bn
    with_prev = prev is not None

    def body(h_ref, x_ref, g_ref, b_ref, *refs):
        o_ref = refs[-1]
        s = jax.nn.sigmoid(h_ref[...]) + x_ref[...]
        mu = jnp.mean(s, axis=1, keepdims=True)
        c = s - mu
        var = jnp.mean(c * c, axis=1, keepdims=True)
        o_ref[...] = c * lax.rsqrt(var + 1e-5) * g_ref[...] + b_ref[...]

    in_specs = [
        pl.BlockSpec((bn, D), lambda i: (i, 0)),
        pl.BlockSpec((bn, D), lambda i: (i, 0)),
        pl.BlockSpec((1, D), lambda i: (0, 0)),
        pl.BlockSpec((1, D), lambda i: (0, 0)),
    ]
    args = [hp, x, g.reshape(1, D), b.reshape(1, D)]
    if with_prev:
        in_specs.append(pl.BlockSpec((bn, D), lambda i, o=blk_off: (i + o, 0)))
        args.append(prev)
    return pl.pallas_call(
        body,
        grid=(n // bn,),
        in_specs=in_specs,
        out_specs=pl.BlockSpec((bn, D), lambda i, o=blk_off: (i + o, 0)),
        out_shape=jax.ShapeDtypeStruct((NTOT, D), _f32),
        input_output_aliases={4: 0} if with_prev else {},
    )(*args)


def _sc_spmm_body(
    t_s0, t_h0, t_s1, t_l1, t_h1, t_s2, t_l2,
    l0r, l0c, l0v, i1r, i1c, i1v, l1r, l1c, l1v,
    i2r, i2c, i2v, l2r, l2c, l2v,
    h0_out, h1_out, h2_out,
    acc, rowb, colb, valb, lrow_s, col_s, val_s,
    sidx0, sidx1, sidx2, sidx3, srow0, srow1, srow2, srow3,
    gbuf, zbuf, ns_ref, sem_g, sem_w,
):
    cid = lax.axis_index("c")
    sid = lax.axis_index("s")
    sidx = (sidx0, sidx1, sidx2, sidx3)
    srow = (srow0, srow1, srow2, srow3)

    def _vcopy(src, off, dst):
        for q in range(128 // 16):
            dst[pl.ds(q * 16, 16)] = src[pl.ds(off + q * 16, 16)]

    def _zrow(r, _):
        for q in range(D // 16):
            zbuf[r, pl.ds(q * 16, 16)] = jnp.zeros((16,), _f32)
        return _

    lax.fori_loop(0, ZR, _zrow, None)

    def _zstage(i, _):
        z16i = jnp.zeros((16,), _i32)
        lrow_s[pl.ds(i * 16, 16)] = z16i
        col_s[pl.ds(i * 16, 16)] = z16i
        val_s[pl.ds(i * 16, 16)] = jnp.zeros((16,), _f32)
        return _

    lax.fori_loop(0, K // 16, _zstage, None)
    ns_ref[0] = 0

    def _zero_my_slice():
        for z in range(RPT // ZR):
            pltpu.sync_copy(zbuf, acc.at[pl.ds(sid * RPT + z * ZR, ZR)])

    _zero_my_slice()
    plsc.subcore_barrier()

    def _scale_rows(n_rows):
        iota16 = lax.iota(_i32, 16)
        n_grp = (n_rows + 15) // 16

        def grp_body(gi, _):
            vv = val_s[pl.ds(gi * 16, 16)]
            rvec = gi * 16 + iota16

            def q_body(qi, _):
                for u in range(8):
                    cvec = (iota16 + (qi * 8 + u)) & (D - 1)
                    col = plsc.load_gather(gbuf, [rvec, cvec])
                    plsc.store_scatter(gbuf, [rvec, cvec], col * vv)
                return _

            lax.fori_loop(0, D // 8, q_body, None)
            return _

        lax.fori_loop(0, n_grp, grp_body, None)

    def _reset_stage():
        def zb(i, _):
            val_s[pl.ds(i * 16, 16)] = jnp.zeros((16,), _f32)
            return _

        lax.fori_loop(0, K // 16, zb, None)
        ns_ref[0] = 0

    def _flush_full(tbl, acc_ref):
        descs = []
        for j in range(KB):
            _vcopy(col_s, j * 128, sidx[j])
            descs.append(
                pltpu.async_copy(
                    tbl.at[sidx[j]], gbuf.at[pl.ds(j * 128, 128)], sem_g
                )
            )
        for d in descs:
            d.wait()
        _scale_rows(K)
        for j in range(KB):
            _vcopy(lrow_s, j * 128, srow[j])
            pltpu.sync_copy(
                gbuf.at[pl.ds(j * 128, 128)], acc_ref.at[srow[j]], add=True
            )
        _reset_stage()

    def _flush_tail(tbl, acc_ref):
        n = ns_ref[0]

        @pl.when(n > 0)
        def _():
            for j in range(KB):
                @pl.when(n > j * 128)
                def _():
                    _vcopy(col_s, j * 128, sidx[j])
                    pltpu.sync_copy(
                        tbl.at[sidx[j]], gbuf.at[pl.ds(j * 128, 128)]
                    )
            _scale_rows(((n + 127) // 128) * 128)
            for j in range(KB):
                @pl.when(n > j * 128)
                def _():
                    _vcopy(lrow_s, j * 128, srow[j])
                    pltpu.sync_copy(
                        gbuf.at[pl.ds(j * 128, 128)],
                        acc_ref.at[srow[j]],
                        add=True,
                    )
            _reset_stage()

    def _scan_vregs(row0, n_vregs, tbl, acc_ref):
        def _hit(r, cvec, vvec, m, cnt, do_flush):
            ns = ns_ref[0]
            s = plsc.cumsum(jnp.where(m, 1, 0).astype(_i32))
            pos = s + (ns - 1)
            plsc.store_scatter(lrow_s, [pos], r - row0, mask=m)
            plsc.store_scatter(col_s, [pos], cvec, mask=m)
            plsc.store_scatter(val_s, [pos], vvec, mask=m)
            ns_ref[0] = ns + cnt

            @pl.when(ns_ref[0] >= K - 16)
            def _():
                do_flush()

        def vbody(v, _):
            b = v * 32
            r0v = rowb[pl.ds(b, 16)]
            r1v = rowb[pl.ds(b + 16, 16)]
            m0 = (r0v >= row0) & (r0v < row0 + CHUNK)
            m1 = (r1v >= row0) & (r1v < row0 + CHUNK)
            c0 = plsc.all_reduce_population_count(m0)
            c1 = plsc.all_reduce_population_count(m1)

            @pl.when(c0[0] > 0)
            def _():
                _hit(r0v, colb[pl.ds(b, 16)], valb[pl.ds(b, 16)], m0, c0[0],
                     lambda: _flush_full(tbl, acc_ref))

            @pl.when(c1[0] > 0)
            def _():
                _hit(r1v, colb[pl.ds(b + 16, 16)], valb[pl.ds(b + 16, 16)],
                     m1, c1[0], lambda: _flush_full(tbl, acc_ref))

            return _

        lax.fori_loop(0, n_vregs // 2, vbody, None)

    def _process_triplet(row0, acc_ref, rr, cc, vv, nnz, tbl):
        full_w = nnz // W
        n_win_me = (full_w - sid + 15) // 16

        def win_body(k, _):
            off = (k * 16 + sid) * W
            d0 = pltpu.async_copy(rr.at[pl.ds(off, W)], rowb, sem_w)
            d1 = pltpu.async_copy(cc.at[pl.ds(off, W)], colb, sem_w)
            d2 = pltpu.async_copy(vv.at[pl.ds(off, W)], valb, sem_w)
            d0.wait()
            d1.wait()
            d2.wait()
            _scan_vregs(row0, W // 16, tbl, acc_ref)
            return _

        lax.fori_loop(0, n_win_me, win_body, None)
        _flush_tail(tbl, acc_ref)

    def _process_rank(out_hbm, n_pad, triplets):
        c_total = n_pad // CHUNK
        c0 = (c_total + 1) // 2
        n_me = jnp.where(cid == 0, c0, c_total - c0)
        base_c = jnp.where(cid == 0, 0, c0)
        n_max = c0

        def chunk_body(i, _):
            @pl.when(i < n_me)
            def _():
                c = base_c + i
                row0 = c * CHUNK
                for (rr, cc, vv, nnz, tbl) in triplets:
                    _process_triplet(row0, acc, rr, cc, vv, nnz, tbl)
                plsc.subcore_barrier()
                pltpu.sync_copy(
                    acc.at[pl.ds(sid * RPT, RPT)],
                    out_hbm.at[pl.ds(row0 + sid * RPT, RPT)],
                )
                _zero_my_slice()
                plsc.subcore_barrier()

            return _

        lax.fori_loop(0, n_max, chunk_body, None)

    _process_rank(
        h0_out, N0P,
        [(l0r, l0c, l0v, NNZ_L0, t_s0), (i1r, i1c, i1v, NNZ_I1, t_h0)],
    )
    _process_rank(
        h1_out, N1P,
        [
            (l1r, l1c, l1v, NNZ_L1, t_s1),
            (i1c, i1r, i1v, NNZ_I1, t_l1),
            (i2r, i2c, i2v, NNZ_I2, t_h1),
        ],
    )
    _process_rank(
        h2_out, N2P,
        [(l2r, l2c, l2v, NNZ_L2, t_s2), (i2c, i2r, i2v, NNZ_I2, t_l2)],
    )


_sc_spmm = pl.kernel(
    _sc_spmm_body,
    out_type=[
        jax.ShapeDtypeStruct((N0P, D), _f32),
        jax.ShapeDtypeStruct((N1P, D), _f32),
        jax.ShapeDtypeStruct((N2P, D), _f32),
    ],
    mesh=plsc.VectorSubcoreMesh(
        core_axis_name="c", subcore_axis_name="s", num_cores=NC, num_subcores=NS
    ),
    compiler_params=pltpu.CompilerParams(needs_layout_passes=False),
    scratch_types=[
        pltpu.VMEM_SHARED((CHUNK, D), _f32),
        pltpu.VMEM((W,), _i32),
        pltpu.VMEM((W,), _i32),
        pltpu.VMEM((W,), _f32),
        pltpu.VMEM((K,), _i32),
        pltpu.VMEM((K,), _i32),
        pltpu.VMEM((K,), _f32),
        pltpu.VMEM((128,), _i32),
        pltpu.VMEM((128,), _i32),
        pltpu.VMEM((128,), _i32),
        pltpu.VMEM((128,), _i32),
        pltpu.VMEM((128,), _i32),
        pltpu.VMEM((128,), _i32),
        pltpu.VMEM((128,), _i32),
        pltpu.VMEM((128,), _i32),
        pltpu.VMEM((K, D), _f32),
        pltpu.VMEM((ZR, D), _f32),
        pltpu.SMEM((1,), _i32),
        pltpu.SemaphoreType.DMA,
        pltpu.SemaphoreType.DMA,
    ],
)


def kernel(x_0, x_1, x_2, inc1_rows, inc1_cols, inc1_vals, inc2_rows, inc2_cols, inc2_vals, L0_rows, L0_cols, L0_vals, L1_rows, L1_cols, L1_vals, L2_rows, L2_cols, L2_vals, W_same_0, W_same_1, W_same_2, W_low_1, W_low_2, W_high_0, W_high_1, g0, b0, g1, b1, g2, b2, y, batch_0):
    t_s0, t_l1 = _mm_multi(x_0, [W_same_0, W_low_1], 2000)
    t_s1, t_h0, t_l2 = _mm_multi(x_1, [W_same_1, W_high_0, W_low_2], 2000)
    t_s2, t_h1 = _mm_multi(x_2, [W_same_2, W_high_1], 2000)

    def _pad_coo(r, c, v):
        npad = _wpad(r.shape[0]) - r.shape[0]
        if npad == 0:
            return r, c, v
        return (
            jnp.concatenate([r, jnp.full((npad,), -1, r.dtype)]),
            jnp.concatenate([c, jnp.full((npad,), -1, c.dtype)]),
            jnp.concatenate([v, jnp.zeros((npad,), v.dtype)]),
        )

    l0 = _pad_coo(L0_rows, L0_cols, L0_vals)
    i1 = _pad_coo(inc1_rows, inc1_cols, inc1_vals)
    l1 = _pad_coo(L1_rows, L1_cols, L1_vals)
    i2 = _pad_coo(inc2_rows, inc2_cols, inc2_vals)
    l2 = _pad_coo(L2_rows, L2_cols, L2_vals)

    h0p, h1p, h2p = _sc_spmm(
        t_s0, t_h0, t_s1, t_l1, t_h1, t_s2, t_l2,
        *l0, *i1, *l1, *i2, *l2,
    )

    out = _ln_into(h0p, x_0, g0, b0, None, 0, 2000)
    out = _ln_into(h1p, x_1, g1, b1, out, N0, 2000)
    out = _ln_into(h2p, x_2, g2, b2, out, N0 + N1, 2000)
    return out
